# baseline (device time: 183794 ns/iter reference)
import jax
import jax.numpy as jnp
from jax import lax
from jax.experimental import pallas as pl
from jax.experimental.pallas import tpu as pltpu

N_DEV = 8


def _gelu(z):
    return 0.5 * z * (1.0 + jnp.tanh(0.7978845608 * (z + 0.044715 * z * z * z)))


def kernel(A, B):
    m, k = A.shape
    k2, n = B.shape

    def body(a_ref, b_ref, out_ref, comm_ref, send_sems, recv_sems):
        my_pos = lax.axis_index("i")
        left = (my_pos - 1) % N_DEV
        right = (my_pos + 1) % N_DEV

        barrier_sem = pltpu.get_barrier_semaphore()
        for nbr in [left, right]:
            pl.semaphore_signal(
                barrier_sem, inc=1,
                device_id=(nbr,), device_id_type=pl.DeviceIdType.MESH,
            )
        pl.semaphore_wait(barrier_sem, 2)

        partial = jnp.dot(
            a_ref[:, :].astype(jnp.bfloat16),
            b_ref[:, :].astype(jnp.bfloat16),
            preferred_element_type=jnp.float32,
        )
        out_ref[:, :] = partial
        comm_ref[0, :, :] = partial.astype(jnp.bfloat16)

        for h in range(N_DEV - 1):
            rdma = pltpu.make_async_remote_copy(
                src_ref=comm_ref.at[h],
                dst_ref=comm_ref.at[h + 1],
                send_sem=send_sems.at[h],
                recv_sem=recv_sems.at[h],
                device_id=(right,),
                device_id_type=pl.DeviceIdType.MESH,
            )
            rdma.start()
            rdma.wait()
            out_ref[:, :] += comm_ref[h + 1, :, :].astype(jnp.float32)

        out_ref[:, :] = _gelu(out_ref[:, :])

    return pl.pallas_call(
        body,
        out_shape=jax.ShapeDtypeStruct((m, n), jnp.float32),
        in_specs=[
            pl.BlockSpec(memory_space=pltpu.VMEM),
            pl.BlockSpec(memory_space=pltpu.VMEM),
        ],
        out_specs=pl.BlockSpec(memory_space=pltpu.VMEM),
        scratch_shapes=[
            pltpu.VMEM((N_DEV, m, n), jnp.bfloat16),
            pltpu.SemaphoreType.DMA((N_DEV - 1,)),
            pltpu.SemaphoreType.DMA((N_DEV - 1,)),
        ],
        compiler_params=pltpu.CompilerParams(collective_id=0),
    )(A, B)


# device time: 43238 ns/iter; 4.2508x vs baseline; 4.2508x over previous
import jax
import jax.numpy as jnp
from jax import lax
from jax.experimental import pallas as pl
from jax.experimental.pallas import tpu as pltpu

N_DEV = 8


def _gelu(z):
    return 0.5 * z * (1.0 + jnp.tanh(0.7978845608 * (z + 0.044715 * z * z * z)))


def kernel(A, B):
    m, k = A.shape
    k2, n = B.shape
    rows = m // N_DEV

    def body(a_ref, b_ref, out_ref,
             src_ref, rs_ref, ag_ref,
             rs_send_sems, rs_recv_sems, ag_send_sems, ag_recv_sems):
        my_pos = lax.axis_index("i")

        barrier_sem = pltpu.get_barrier_semaphore()
        for d in range(1, N_DEV):
            peer = lax.rem(my_pos + d, N_DEV)
            pl.semaphore_signal(
                barrier_sem, inc=1,
                device_id=(peer,), device_id_type=pl.DeviceIdType.MESH,
            )
        pl.semaphore_wait(barrier_sem, N_DEV - 1)

        partial = jnp.dot(
            a_ref[:, :].astype(jnp.bfloat16),
            b_ref[:, :].astype(jnp.bfloat16),
            preferred_element_type=jnp.float32,
        )
        for j in range(N_DEV):
            src_ref[j, :, :] = partial[j * rows:(j + 1) * rows, :].astype(
                jnp.bfloat16)

        rs_rdmas = []
        for d in range(1, N_DEV):
            j = lax.rem(my_pos + d, N_DEV)
            rdma = pltpu.make_async_remote_copy(
                src_ref=src_ref.at[j],
                dst_ref=rs_ref.at[my_pos],
                send_sem=rs_send_sems.at[j],
                recv_sem=rs_recv_sems.at[my_pos],
                device_id=(j,),
                device_id_type=pl.DeviceIdType.MESH,
            )
            rdma.start()
            rs_rdmas.append(rdma)
        rs_ref[my_pos, :, :] = src_ref[my_pos, :, :]

        for d in range(1, N_DEV):
            i = lax.rem(my_pos + d, N_DEV)
            recv = pltpu.make_async_remote_copy(
                src_ref=src_ref.at[i],
                dst_ref=rs_ref.at[i],
                send_sem=rs_send_sems.at[i],
                recv_sem=rs_recv_sems.at[i],
                device_id=(i,),
                device_id_type=pl.DeviceIdType.MESH,
            )
            recv.wait_recv()

        acc = rs_ref[0, :, :].astype(jnp.float32)
        for i in range(1, N_DEV):
            acc += rs_ref[i, :, :].astype(jnp.float32)
        ag_ref[my_pos, :, :] = _gelu(acc).astype(jnp.bfloat16)

        ag_rdmas = []
        for d in range(1, N_DEV):
            j = lax.rem(my_pos + d, N_DEV)
            rdma = pltpu.make_async_remote_copy(
                src_ref=ag_ref.at[my_pos],
                dst_ref=ag_ref.at[my_pos],
                send_sem=ag_send_sems.at[j],
                recv_sem=ag_recv_sems.at[my_pos],
                device_id=(j,),
                device_id_type=pl.DeviceIdType.MESH,
            )
            rdma.start()
            ag_rdmas.append(rdma)
        for d in range(1, N_DEV):
            i = lax.rem(my_pos + d, N_DEV)
            recv = pltpu.make_async_remote_copy(
                src_ref=ag_ref.at[i],
                dst_ref=ag_ref.at[i],
                send_sem=ag_send_sems.at[i],
                recv_sem=ag_recv_sems.at[i],
                device_id=(i,),
                device_id_type=pl.DeviceIdType.MESH,
            )
            recv.wait_recv()

        for i in range(N_DEV):
            out_ref[i * rows:(i + 1) * rows, :] = ag_ref[i, :, :].astype(
                jnp.float32)

        for rdma in rs_rdmas + ag_rdmas:
            rdma.wait_send()

    return pl.pallas_call(
        body,
        out_shape=jax.ShapeDtypeStruct((m, n), jnp.float32),
        in_specs=[
            pl.BlockSpec(memory_space=pltpu.VMEM),
            pl.BlockSpec(memory_space=pltpu.VMEM),
        ],
        out_specs=pl.BlockSpec(memory_space=pltpu.VMEM),
        scratch_shapes=[
            pltpu.VMEM((N_DEV, rows, n), jnp.bfloat16),
            pltpu.VMEM((N_DEV, rows, n), jnp.bfloat16),
            pltpu.VMEM((N_DEV, rows, n), jnp.bfloat16),
            pltpu.SemaphoreType.DMA((N_DEV,)),
            pltpu.SemaphoreType.DMA((N_DEV,)),
            pltpu.SemaphoreType.DMA((N_DEV,)),
            pltpu.SemaphoreType.DMA((N_DEV,)),
        ],
        compiler_params=pltpu.CompilerParams(collective_id=0),
    )(A, B)


# device time: 34747 ns/iter; 5.2895x vs baseline; 1.2444x over previous
import jax
import jax.numpy as jnp
from jax import lax
from jax.experimental import pallas as pl
from jax.experimental.pallas import tpu as pltpu

N_DEV = 8
S = 2


def _gelu(z):
    return 0.5 * z * (1.0 + jnp.tanh(0.7978845608 * (z + 0.044715 * z * z * z)))


def kernel(A, B):
    m, k = A.shape
    k2, n = B.shape
    rows = m // N_DEV
    srows = rows // S

    def body(a_ref, b_ref, out_ref,
             src_ref, rs_ref, ag_ref,
             rs_send_sems, rs_recv_sems, ag_send_sems, ag_recv_sems):
        my_pos = lax.axis_index("i")

        barrier_sem = pltpu.get_barrier_semaphore()
        for d in range(1, N_DEV):
            peer = lax.rem(my_pos + d, N_DEV)
            pl.semaphore_signal(
                barrier_sem, inc=1,
                device_id=(peer,), device_id_type=pl.DeviceIdType.MESH,
            )

        partial = jnp.dot(
            a_ref[:, :].astype(jnp.bfloat16),
            b_ref[:, :].astype(jnp.bfloat16),
            preferred_element_type=jnp.float32,
        )
        for s in range(S):
            for j in range(N_DEV):
                r0 = j * rows + s * srows
                src_ref[s, j, :, :] = partial[r0:r0 + srows, :].astype(
                    jnp.bfloat16)

        pl.semaphore_wait(barrier_sem, N_DEV - 1)

        send_rdmas = []

        def rs_descriptor(s, i):
            return pltpu.make_async_remote_copy(
                src_ref=src_ref.at[s, i],
                dst_ref=rs_ref.at[s, i],
                send_sem=rs_send_sems.at[s, i],
                recv_sem=rs_recv_sems.at[s, i],
                device_id=(i,),
                device_id_type=pl.DeviceIdType.MESH,
            )

        def ag_descriptor(s, i):
            return pltpu.make_async_remote_copy(
                src_ref=ag_ref.at[s, i],
                dst_ref=ag_ref.at[s, i],
                send_sem=ag_send_sems.at[s, i],
                recv_sem=ag_recv_sems.at[s, i],
                device_id=(i,),
                device_id_type=pl.DeviceIdType.MESH,
            )

        for s in range(S):
            for d in range(1, N_DEV):
                j = lax.rem(my_pos + d, N_DEV)
                rdma = pltpu.make_async_remote_copy(
                    src_ref=src_ref.at[s, j],
                    dst_ref=rs_ref.at[s, my_pos],
                    send_sem=rs_send_sems.at[s, j],
                    recv_sem=rs_recv_sems.at[s, my_pos],
                    device_id=(j,),
                    device_id_type=pl.DeviceIdType.MESH,
                )
                rdma.start()
                send_rdmas.append(rdma)
            rs_ref[s, my_pos, :, :] = src_ref[s, my_pos, :, :]

        for s in range(S):
            for d in range(1, N_DEV):
                i = lax.rem(my_pos + d, N_DEV)
                rs_descriptor(s, i).wait_recv()
            acc = rs_ref[s, 0, :, :].astype(jnp.float32)
            for i in range(1, N_DEV):
                acc += rs_ref[s, i, :, :].astype(jnp.float32)
            ag_ref[s, my_pos, :, :] = _gelu(acc).astype(jnp.bfloat16)
            for d in range(1, N_DEV):
                j = lax.rem(my_pos + d, N_DEV)
                rdma = pltpu.make_async_remote_copy(
                    src_ref=ag_ref.at[s, my_pos],
                    dst_ref=ag_ref.at[s, my_pos],
                    send_sem=ag_send_sems.at[s, j],
                    recv_sem=ag_recv_sems.at[s, my_pos],
                    device_id=(j,),
                    device_id_type=pl.DeviceIdType.MESH,
                )
                rdma.start()
                send_rdmas.append(rdma)

        for s in range(S):
            for d in range(1, N_DEV):
                i = lax.rem(my_pos + d, N_DEV)
                ag_descriptor(s, i).wait_recv()
            for i in range(N_DEV):
                r0 = i * rows + s * srows
                out_ref[r0:r0 + srows, :] = ag_ref[s, i, :, :].astype(
                    jnp.float32)

        for rdma in send_rdmas:
            rdma.wait_send()

    return pl.pallas_call(
        body,
        out_shape=jax.ShapeDtypeStruct((m, n), jnp.float32),
        in_specs=[
            pl.BlockSpec(memory_space=pltpu.VMEM),
            pl.BlockSpec(memory_space=pltpu.VMEM),
        ],
        out_specs=pl.BlockSpec(memory_space=pltpu.VMEM),
        scratch_shapes=[
            pltpu.VMEM((S, N_DEV, srows, n), jnp.bfloat16),
            pltpu.VMEM((S, N_DEV, srows, n), jnp.bfloat16),
            pltpu.VMEM((S, N_DEV, srows, n), jnp.bfloat16),
            pltpu.SemaphoreType.DMA((S, N_DEV)),
            pltpu.SemaphoreType.DMA((S, N_DEV)),
            pltpu.SemaphoreType.DMA((S, N_DEV)),
            pltpu.SemaphoreType.DMA((S, N_DEV)),
        ],
        compiler_params=pltpu.CompilerParams(collective_id=0),
    )(A, B)
